# baseline (speedup 1.0000x reference)
import functools

import jax
import jax.numpy as jnp
from jax import lax
from jax.experimental import pallas as pl
from jax.experimental.pallas import tpu as pltpu
from jax.experimental.pallas import tpu_sc as plsc

LANES = 16
P = 4


def _sc_kernel(x, pos_table):
    seq_len, batch, d_model = x.shape
    info = plsc.get_sparse_core_info()
    nw = info.num_cores * info.num_subcores
    spw = seq_len // nw
    nch = spw // P

    mesh = plsc.VectorSubcoreMesh(core_axis_name="c", subcore_axis_name="s")

    @functools.partial(
        pl.kernel,
        mesh=mesh,
        out_type=jax.ShapeDtypeStruct((seq_len, batch, d_model), x.dtype),
        scratch_types=[
            pltpu.VMEM((2, P, batch, d_model), jnp.float32),
            pltpu.VMEM((2, P, d_model), jnp.float32),
            pltpu.VMEM((2, P, batch, d_model), jnp.float32),
            pltpu.SemaphoreType.DMA,
            pltpu.SemaphoreType.DMA,
            pltpu.SemaphoreType.DMA,
            pltpu.SemaphoreType.DMA,
            pltpu.SemaphoreType.DMA,
            pltpu.SemaphoreType.DMA,
        ],
    )
    def sc_add(x_hbm, pos_hbm, out_hbm, xin, pin, xout,
               sx0, sx1, sp0, sp1, so0, so1):
        wid = lax.axis_index("s") * info.num_cores + lax.axis_index("c")
        base = wid * spw
        sx = (sx0, sx1)
        sp = (sp0, sp1)
        so = (so0, so1)

        def issue_in(i):
            s = i % 2
            s0 = base + i * P
            cx = pltpu.async_copy(x_hbm.at[pl.ds(s0, P)], xin.at[s], sx[s])
            cp = pltpu.async_copy(pos_hbm.at[pl.ds(s0, P)], pin.at[s], sp[s])
            return cx, cp

        handles_in = {0: issue_in(0), 1: issue_in(1)}
        handles_out = {}
        for i in range(nch):
            s = i % 2
            cx, cp = handles_in.pop(i)
            cx.wait()
            cp.wait()
            if i >= 2:
                handles_out.pop(i - 2).wait()

            def body(j, _):
                for p in range(P):
                    pv = pin[s, p, pl.ds(j * LANES, LANES)]
                    for b in range(batch):
                        xout[s, p, b, pl.ds(j * LANES, LANES)] = (
                            xin[s, p, b, pl.ds(j * LANES, LANES)] + pv
                        )
                return 0

            lax.fori_loop(0, d_model // LANES, body, 0)
            s0 = base + i * P
            handles_out[i] = pltpu.async_copy(
                xout.at[s], out_hbm.at[pl.ds(s0, P)], so[s]
            )
            if i + 2 < nch:
                handles_in[i + 2] = issue_in(i + 2)
        for i in (nch - 2, nch - 1):
            handles_out.pop(i).wait()

    return sc_add(x, pos_table[:seq_len])


S_BLK = 512


def _tc_body(x_ref, pos_ref, out_ref):
    pos = pos_ref[...]
    for b in range(x_ref.shape[1]):
        out_ref[:, b, :] = x_ref[:, b, :] + pos


def _tc_kernel(x, pos_table):
    seq_len, batch, d_model = x.shape
    grid = (seq_len // S_BLK,)
    return pl.pallas_call(
        _tc_body,
        grid=grid,
        in_specs=[
            pl.BlockSpec((S_BLK, batch, d_model), lambda i: (i, 0, 0)),
            pl.BlockSpec((S_BLK, d_model), lambda i: (i, 0)),
        ],
        out_specs=pl.BlockSpec((S_BLK, batch, d_model), lambda i: (i, 0, 0)),
        out_shape=jax.ShapeDtypeStruct((seq_len, batch, d_model), x.dtype),
        compiler_params=pltpu.CompilerParams(
            dimension_semantics=("arbitrary",),
        ),
    )(x, pos_table[:seq_len])


def kernel(x, pos_table):
    return _sc_kernel(x, pos_table)


# speedup vs baseline: 1.0081x; 1.0081x over previous
import functools

import jax
import jax.numpy as jnp
from jax import lax
from jax.experimental import pallas as pl
from jax.experimental.pallas import tpu as pltpu
from jax.experimental.pallas import tpu_sc as plsc

LANES = 16
P = 4


def _sc_kernel(x, pos_table, rows=None):
    seq_len, batch, d_model = x.shape
    if rows is None:
        rows = seq_len
    info = plsc.get_sparse_core_info()
    nw = info.num_cores * info.num_subcores
    spw = rows // nw
    nch = spw // P

    mesh = plsc.VectorSubcoreMesh(core_axis_name="c", subcore_axis_name="s")

    @functools.partial(
        pl.kernel,
        mesh=mesh,
        out_type=jax.ShapeDtypeStruct((rows, batch, d_model), x.dtype),
        scratch_types=[
            pltpu.VMEM((2, P, batch, d_model), jnp.float32),
            pltpu.VMEM((2, P, d_model), jnp.float32),
            pltpu.VMEM((2, P, batch, d_model), jnp.float32),
            pltpu.SemaphoreType.DMA,
            pltpu.SemaphoreType.DMA,
            pltpu.SemaphoreType.DMA,
            pltpu.SemaphoreType.DMA,
            pltpu.SemaphoreType.DMA,
            pltpu.SemaphoreType.DMA,
        ],
    )
    def sc_add(x_hbm, pos_hbm, out_hbm, xin, pin, xout,
               sx0, sx1, sp0, sp1, so0, so1):
        wid = lax.axis_index("s") * info.num_cores + lax.axis_index("c")
        base = wid * spw
        sx = (sx0, sx1)
        sp = (sp0, sp1)
        so = (so0, so1)

        def issue_in(i):
            s = i % 2
            s0 = base + i * P
            cx = pltpu.async_copy(x_hbm.at[pl.ds(s0, P)], xin.at[s], sx[s])
            cp = pltpu.async_copy(pos_hbm.at[pl.ds(s0, P)], pin.at[s], sp[s])
            return cx, cp

        handles_in = {0: issue_in(0), 1: issue_in(1)}
        handles_out = {}
        for i in range(nch):
            s = i % 2
            cx, cp = handles_in.pop(i)
            cx.wait()
            cp.wait()
            if i >= 2:
                handles_out.pop(i - 2).wait()

            def body(j, _):
                for p in range(P):
                    pv = pin[s, p, pl.ds(j * LANES, LANES)]
                    for b in range(batch):
                        xout[s, p, b, pl.ds(j * LANES, LANES)] = (
                            xin[s, p, b, pl.ds(j * LANES, LANES)] + pv
                        )
                return 0

            lax.fori_loop(0, d_model // LANES, body, 0)
            s0 = base + i * P
            handles_out[i] = pltpu.async_copy(
                xout.at[s], out_hbm.at[pl.ds(s0, P)], so[s]
            )
            if i + 2 < nch:
                handles_in[i + 2] = issue_in(i + 2)
        for i in (nch - 2, nch - 1):
            handles_out.pop(i).wait()

    return sc_add(x, pos_table[:seq_len])


S_BLK = 512


def _tc_body(x_ref, pos_ref, out_ref):
    pos = pos_ref[...]
    for b in range(x_ref.shape[1]):
        out_ref[:, b, :] = x_ref[:, b, :] + pos


def _tc_kernel(x, pos_table, start=0, blk=S_BLK):
    seq_len, batch, d_model = x.shape
    off = start // blk
    grid = ((seq_len - start) // blk,)
    return pl.pallas_call(
        _tc_body,
        grid=grid,
        in_specs=[
            pl.BlockSpec((blk, batch, d_model), lambda i: (i + off, 0, 0)),
            pl.BlockSpec((blk, d_model), lambda i: (i + off, 0)),
        ],
        out_specs=pl.BlockSpec((blk, batch, d_model), lambda i: (i + off, 0, 0)),
        out_shape=jax.ShapeDtypeStruct((seq_len, batch, d_model), x.dtype),
        compiler_params=pltpu.CompilerParams(
            dimension_semantics=("arbitrary",),
        ),
    )(x, pos_table[:seq_len])


SC_ROWS = 512


def kernel(x, pos_table):
    sc_part = _sc_kernel(x, pos_table, rows=SC_ROWS)
    tc_full = _tc_kernel(x, pos_table, start=SC_ROWS)
    return lax.dynamic_update_slice(tc_full, sc_part, (0, 0, 0))


# speedup vs baseline: 1.0097x; 1.0016x over previous
import functools

import jax
import jax.numpy as jnp
from jax import lax
from jax.experimental import pallas as pl
from jax.experimental.pallas import tpu as pltpu
from jax.experimental.pallas import tpu_sc as plsc

LANES = 16
P = 4


def _sc_kernel(x, pos_table, rows=None):
    seq_len, batch, d_model = x.shape
    if rows is None:
        rows = seq_len
    info = plsc.get_sparse_core_info()
    nw = info.num_cores * info.num_subcores
    spw = rows // nw
    nch = spw // P

    mesh = plsc.VectorSubcoreMesh(core_axis_name="c", subcore_axis_name="s")

    @functools.partial(
        pl.kernel,
        mesh=mesh,
        out_type=jax.ShapeDtypeStruct((rows, batch, d_model), x.dtype),
        scratch_types=[
            pltpu.VMEM((2, P, batch, d_model), jnp.float32),
            pltpu.VMEM((2, P, d_model), jnp.float32),
            pltpu.VMEM((2, P, batch, d_model), jnp.float32),
            pltpu.SemaphoreType.DMA,
            pltpu.SemaphoreType.DMA,
            pltpu.SemaphoreType.DMA,
            pltpu.SemaphoreType.DMA,
            pltpu.SemaphoreType.DMA,
            pltpu.SemaphoreType.DMA,
        ],
    )
    def sc_add(x_hbm, pos_hbm, out_hbm, xin, pin, xout,
               sx0, sx1, sp0, sp1, so0, so1):
        wid = lax.axis_index("s") * info.num_cores + lax.axis_index("c")
        base = wid * spw
        sx = (sx0, sx1)
        sp = (sp0, sp1)
        so = (so0, so1)

        def issue_in(i):
            s = i % 2
            s0 = base + i * P
            cx = pltpu.async_copy(x_hbm.at[pl.ds(s0, P)], xin.at[s], sx[s])
            cp = pltpu.async_copy(pos_hbm.at[pl.ds(s0, P)], pin.at[s], sp[s])
            return cx, cp

        handles_in = {0: issue_in(0), 1: issue_in(1)}
        handles_out = {}
        for i in range(nch):
            s = i % 2
            cx, cp = handles_in.pop(i)
            cx.wait()
            cp.wait()
            if i >= 2:
                handles_out.pop(i - 2).wait()

            def body(j, _):
                for p in range(P):
                    pv = pin[s, p, pl.ds(j * LANES, LANES)]
                    for b in range(batch):
                        xout[s, p, b, pl.ds(j * LANES, LANES)] = (
                            xin[s, p, b, pl.ds(j * LANES, LANES)] + pv
                        )
                return 0

            lax.fori_loop(0, d_model // LANES, body, 0)
            s0 = base + i * P
            handles_out[i] = pltpu.async_copy(
                xout.at[s], out_hbm.at[pl.ds(s0, P)], so[s]
            )
            if i + 2 < nch:
                handles_in[i + 2] = issue_in(i + 2)
        for i in (nch - 2, nch - 1):
            handles_out.pop(i).wait()

    return sc_add(x, pos_table[:seq_len])


S_BLK = 512


def _tc_body(x_ref, pos_ref, out_ref):
    pos = pos_ref[...]
    for b in range(x_ref.shape[1]):
        out_ref[:, b, :] = x_ref[:, b, :] + pos


def _tc_kernel(x, pos_table, start=0, blk=S_BLK):
    seq_len, batch, d_model = x.shape
    off = start // blk
    grid = ((seq_len - start) // blk,)
    return pl.pallas_call(
        _tc_body,
        grid=grid,
        in_specs=[
            pl.BlockSpec((blk, batch, d_model), lambda i: (i + off, 0, 0)),
            pl.BlockSpec((blk, d_model), lambda i: (i + off, 0)),
        ],
        out_specs=pl.BlockSpec((blk, batch, d_model), lambda i: (i + off, 0, 0)),
        out_shape=jax.ShapeDtypeStruct((seq_len, batch, d_model), x.dtype),
        compiler_params=pltpu.CompilerParams(
            dimension_semantics=("arbitrary",),
        ),
    )(x, pos_table[:seq_len])


SC_ROWS = 512


def kernel(x, pos_table):
    tc_full = _tc_kernel(x, pos_table, start=SC_ROWS)
    sc_part = _sc_kernel(x, pos_table, rows=SC_ROWS)
    return lax.dynamic_update_slice(tc_full, sc_part, (0, 0, 0))
